# baseline (speedup 1.0000x reference)
import functools
from itertools import combinations

import numpy as np
import jax
import jax.numpy as jnp
from jax import lax
from jax.experimental import pallas as pl
from jax.experimental.pallas import tpu as pltpu
from jax.experimental.pallas import tpu_sc as plsc

_B, _F, _V, _D, _H = 4096, 26, 100000, 16, 128
_PAIRS = np.array(list(combinations(range(_F), 2)), dtype=np.int32)
_NPAIR = _PAIRS.shape[0]
_PI = _PAIRS[:, 0]
_PJ = _PAIRS[:, 1]

_NC, _NS, _L = 2, 16, 16
_NW = _NC * _NS
_ROWS = _B * _F
_NR = _ROWS // _NW
_NVEC = _NR // _L
_QW = _NR // 8
_CH = 128
_NCHUNK = _NR // _CH
_VC = 12800
_VP = 102400
_VCQ = _VC // 8
_VPK = _VP // 8


@functools.lru_cache(maxsize=None)
def _build_gather():
    mesh = plsc.VectorSubcoreMesh(core_axis_name="c", subcore_axis_name="s")

    @functools.partial(
        pl.kernel,
        mesh=mesh,
        out_type=jax.ShapeDtypeStruct((_ROWS // 8, 128), jnp.float32),
        scratch_types=[
            pltpu.VMEM((_NR,), jnp.int32),
            pltpu.VMEM((_NR,), jnp.int32),
            pltpu.VMEM((2 * _CH, 128), jnp.float32),
            pltpu.VMEM((_QW, 128), jnp.float32),
            pltpu.SemaphoreType.DMA,
            pltpu.SemaphoreType.DMA,
        ],
    )
    def gather_k(xi_hbm, tpk_hbm, out_hbm, idx_v, lane_v, stg_v, pack_v,
                 sem0, sem1):
        wid = lax.axis_index("s") * _NC + lax.axis_index("c")
        base = wid * _NR
        pltpu.sync_copy(xi_hbm.at[pl.ds(base, _NR)], idx_v)

        def ibody(k, carry):
            sl = pl.ds(k * _L, _L)
            pos = k * _L + lax.iota(jnp.int32, _L)
            fld = lax.rem(pos, _F)
            v = idx_v[sl]
            chunk = lax.div(v, _VC)
            w = v - chunk * _VC
            r = lax.div(w, _VCQ)
            idx_v[sl] = fld * _VPK + chunk * _VCQ + (w - r * _VCQ)
            lane_v[sl] = r * _D
            return carry

        lax.fori_loop(0, _NVEC, ibody, 0)

        def fire(c, buf, sem):
            pltpu.async_copy(
                tpk_hbm.at[idx_v.at[pl.ds(c * _CH, _CH)]],
                stg_v.at[pl.ds(buf * _CH, _CH)], sem)

        def wait_sel(c, buf, sem):
            pltpu.make_async_copy(
                tpk_hbm.at[idx_v.at[pl.ds(c * _CH, _CH)]],
                stg_v.at[pl.ds(buf * _CH, _CH)], sem).wait()
            for g in range(_CH // _L):
                row0 = c * _CH + g * _L
                lvec = lane_v[pl.ds(row0, _L)]
                qbase = lax.div(row0, 8)
                for j in range(_L):
                    s = lvec[j]
                    vals = stg_v[buf * _CH + g * _L + j, pl.ds(s, _D)]
                    pack_v[qbase + j // 8, pl.ds((j % 8) * _D, _D)] = vals

        fire(0, 0, sem0)

        def cbody(c, carry):
            @pl.when(lax.rem(c, 2) == 1)
            def _():
                fire(c, 1, sem1)
                wait_sel(c - 1, 0, sem0)

            @pl.when(lax.rem(c, 2) == 0)
            def _():
                fire(c, 0, sem0)
                wait_sel(c - 1, 1, sem1)

            return carry

        lax.fori_loop(1, _NCHUNK, cbody, 0)
        wait_sel(_NCHUNK - 1, (_NCHUNK - 1) % 2, (sem1, sem0)[_NCHUNK % 2])
        pltpu.sync_copy(pack_v, out_hbm.at[pl.ds(wid * _QW, _QW)])

    return gather_k


def _dense_body(e_ref, xvr_ref, xv_ref, M_ref, W1_ref, b1_ref, w2h_ref,
                b2_ref, o_ref):
    es = e_ref[...] * xvr_ref[...]
    a = jnp.dot(es, M_ref[...], preferred_element_type=jnp.float32)
    q = jnp.sum(es * a, axis=1, keepdims=True)
    h = jnp.dot(xv_ref[...], W1_ref[...], preferred_element_type=jnp.float32)
    h = jnp.maximum(h + b1_ref[...], 0.0)
    o_ref[...] = q + jnp.sum(h * w2h_ref[...], axis=1, keepdims=True) \
        + b2_ref[...]


_BLK = 512
_FD = _F * _D
def _tr_body(tv_ref, o_ref):
    z = tv_ref[0].T
    o_ref[0] = jnp.concatenate(
        [z[r * _VCQ:(r + 1) * _VCQ] for r in range(8)], axis=1)


@functools.lru_cache(maxsize=None)
def _build_transpose():
    return pl.pallas_call(
        _tr_body,
        grid=(_F, _VP // _VC),
        in_specs=[pl.BlockSpec((1, _D, _VC), lambda f, j: (f, 0, j))],
        out_specs=pl.BlockSpec((1, _VCQ, 128), lambda f, j: (f, j, 0)),
        out_shape=jax.ShapeDtypeStruct((_F, _VPK, 128), jnp.float32),
    )


@functools.lru_cache(maxsize=None)
def _build_dense():
    return pl.pallas_call(
        _dense_body,
        grid=(_B // _BLK,),
        in_specs=[
            pl.BlockSpec((_BLK, _FD), lambda i: (i, 0)),
            pl.BlockSpec((_BLK, _FD), lambda i: (i, 0)),
            pl.BlockSpec((_BLK, _F), lambda i: (i, 0)),
            pl.BlockSpec((_FD, _FD), lambda i: (0, 0)),
            pl.BlockSpec((_F, _H), lambda i: (0, 0)),
            pl.BlockSpec((1, _H), lambda i: (0, 0)),
            pl.BlockSpec((1, _H), lambda i: (0, 0)),
            pl.BlockSpec((1, 1), lambda i: (0, 0)),
        ],
        out_specs=pl.BlockSpec((_BLK, 1), lambda i: (i, 0)),
        out_shape=jax.ShapeDtypeStruct((_B, 1), jnp.float32),
    )


def kernel(xv, xi, tables, W1, b1, W2, b2):
    xi32 = xi.astype(jnp.int32).reshape(_ROWS)
    tv = tables.transpose(0, 2, 1)
    tpk = _build_transpose()(tv).reshape(_F * _VPK, 128)
    e = _build_gather()(xi32, tpk)
    e2 = e.reshape(_B, _FD)
    xvr = jnp.repeat(xv, _D, axis=1)

    w2q = W2[:_NPAIR, 0] * 0.5
    m26 = (jnp.zeros((_F, _F), jnp.float32)
           .at[_PI, _PJ].set(w2q).at[_PJ, _PI].set(w2q))
    m = jnp.kron(m26, jnp.eye(_D, dtype=jnp.float32))

    return _build_dense()(
        e2, xvr, xv, m, W1,
        b1.reshape(1, _H),
        W2[_NPAIR:, 0].reshape(1, _H),
        b2.reshape(1, 1),
    )


# speedup vs baseline: 1.3887x; 1.3887x over previous
import functools
from itertools import combinations

import numpy as np
import jax
import jax.numpy as jnp
from jax import lax
from jax.experimental import pallas as pl
from jax.experimental.pallas import tpu as pltpu
from jax.experimental.pallas import tpu_sc as plsc

_B, _F, _V, _D, _H = 4096, 26, 100000, 16, 128
_PAIRS = np.array(list(combinations(range(_F), 2)), dtype=np.int32)
_NPAIR = _PAIRS.shape[0]
_PI = _PAIRS[:, 0]
_PJ = _PAIRS[:, 1]

_NC, _NS, _L = 2, 16, 16
_NW = _NC * _NS
_ROWS = _B * _F
_NR = _ROWS // _NW
_NVEC = _NR // _L
_QW = _NR // 8
_CH = 128
_NCHUNK = _NR // _CH
_VC = 12800
_VP = 102400
_VCQ = _VC // 8
_VPK = _VP // 8


@functools.lru_cache(maxsize=None)
def _build_gather():
    mesh = plsc.VectorSubcoreMesh(core_axis_name="c", subcore_axis_name="s")

    @functools.partial(
        pl.kernel,
        mesh=mesh,
        out_type=jax.ShapeDtypeStruct((_ROWS // 8, 128), jnp.float32),
        scratch_types=[
            pltpu.VMEM((_NR,), jnp.int32),
            pltpu.VMEM((_NR,), jnp.int32),
            pltpu.VMEM((2 * _CH, 128), jnp.float32),
            pltpu.VMEM((_QW, 128), jnp.float32),
            pltpu.SemaphoreType.DMA,
            pltpu.SemaphoreType.DMA,
        ],
    )
    def gather_k(xi_hbm, tpk_hbm, out_hbm, idx_v, lane_v, stg_v, pack_v,
                 sem0, sem1):
        wid = lax.axis_index("s") * _NC + lax.axis_index("c")
        base = wid * _NR
        pltpu.sync_copy(xi_hbm.at[pl.ds(base, _NR)], idx_v)

        def ibody(k, carry):
            sl = pl.ds(k * _L, _L)
            pos = k * _L + lax.iota(jnp.int32, _L)
            fld = lax.rem(pos, _F)
            v = idx_v[sl]
            chunk = lax.div(v, _VC)
            w = v - chunk * _VC
            r = lax.div(w, _VCQ)
            idx_v[sl] = fld * _VPK + chunk * _VCQ + (w - r * _VCQ)
            lane_v[sl] = r * _D
            return carry

        lax.fori_loop(0, _NVEC, ibody, 0)

        def fire(c, buf, sem):
            pltpu.async_copy(
                tpk_hbm.at[idx_v.at[pl.ds(c * _CH, _CH)]],
                stg_v.at[pl.ds(buf * _CH, _CH)], sem)

        def wait_sel(c, buf, sem):
            pltpu.make_async_copy(
                tpk_hbm.at[idx_v.at[pl.ds(c * _CH, _CH)]],
                stg_v.at[pl.ds(buf * _CH, _CH)], sem).wait()
            for g in range(_CH // _L):
                row0 = c * _CH + g * _L
                lvec = lane_v[pl.ds(row0, _L)]
                qbase = lax.div(row0, 8)
                for j in range(_L):
                    s = lvec[j]
                    vals = stg_v[buf * _CH + g * _L + j, pl.ds(s, _D)]
                    pack_v[qbase + j // 8, pl.ds((j % 8) * _D, _D)] = vals

        fire(0, 0, sem0)

        def cbody(c, carry):
            @pl.when(lax.rem(c, 2) == 1)
            def _():
                fire(c, 1, sem1)
                wait_sel(c - 1, 0, sem0)

            @pl.when(lax.rem(c, 2) == 0)
            def _():
                fire(c, 0, sem0)
                wait_sel(c - 1, 1, sem1)

            return carry

        lax.fori_loop(1, _NCHUNK, cbody, 0)
        wait_sel(_NCHUNK - 1, (_NCHUNK - 1) % 2, (sem1, sem0)[_NCHUNK % 2])
        pltpu.sync_copy(pack_v, out_hbm.at[pl.ds(wid * _QW, _QW)])

    return gather_k


def _dense_body(e_ref, xvr_ref, xv_ref, M_ref, W1_ref, b1_ref, w2h_ref,
                b2_ref, o_ref):
    es = e_ref[...] * xvr_ref[...]
    a = jnp.dot(es, M_ref[...], preferred_element_type=jnp.float32)
    q = jnp.sum(es * a, axis=1, keepdims=True)
    h = jnp.dot(xv_ref[...], W1_ref[...], preferred_element_type=jnp.float32)
    h = jnp.maximum(h + b1_ref[...], 0.0)
    o_ref[...] = q + jnp.sum(h * w2h_ref[...], axis=1, keepdims=True) \
        + b2_ref[...]


_BLK = 512
_FD = _F * _D
def _tr_body(tv_ref, o_ref):
    x = tv_ref[0]
    di = lax.broadcasted_iota(jnp.int32, (_D, _D), 0)
    dj = lax.broadcasted_iota(jnp.int32, (_D, _D), 1)
    eye = (di == dj).astype(jnp.float32)
    z = lax.dot_general(x, eye, (((0,), (0,)), ((), ())),
                        preferred_element_type=jnp.float32)
    ed = lax.broadcasted_iota(jnp.int32, (_D, 128), 0)
    el = lax.broadcasted_iota(jnp.int32, (_D, 128), 1)
    acc = jnp.zeros((_VCQ, 128), jnp.float32)
    for r in range(8):
        er = (el - r * _D == ed).astype(jnp.float32)
        acc = acc + jnp.dot(z[r * _VCQ:(r + 1) * _VCQ], er,
                            preferred_element_type=jnp.float32)
    o_ref[0] = acc


@functools.lru_cache(maxsize=None)
def _build_transpose():
    return pl.pallas_call(
        _tr_body,
        grid=(_F, _VP // _VC),
        in_specs=[pl.BlockSpec((1, _D, _VC), lambda f, j: (f, 0, j))],
        out_specs=pl.BlockSpec((1, _VCQ, 128), lambda f, j: (f, j, 0)),
        out_shape=jax.ShapeDtypeStruct((_F, _VPK, 128), jnp.float32),
    )


@functools.lru_cache(maxsize=None)
def _build_dense():
    return pl.pallas_call(
        _dense_body,
        grid=(_B // _BLK,),
        in_specs=[
            pl.BlockSpec((_BLK, _FD), lambda i: (i, 0)),
            pl.BlockSpec((_BLK, _FD), lambda i: (i, 0)),
            pl.BlockSpec((_BLK, _F), lambda i: (i, 0)),
            pl.BlockSpec((_FD, _FD), lambda i: (0, 0)),
            pl.BlockSpec((_F, _H), lambda i: (0, 0)),
            pl.BlockSpec((1, _H), lambda i: (0, 0)),
            pl.BlockSpec((1, _H), lambda i: (0, 0)),
            pl.BlockSpec((1, 1), lambda i: (0, 0)),
        ],
        out_specs=pl.BlockSpec((_BLK, 1), lambda i: (i, 0)),
        out_shape=jax.ShapeDtypeStruct((_B, 1), jnp.float32),
    )


def kernel(xv, xi, tables, W1, b1, W2, b2):
    xi32 = xi.astype(jnp.int32).reshape(_ROWS)
    tv = tables.transpose(0, 2, 1)
    tpk = _build_transpose()(tv).reshape(_F * _VPK, 128)
    e = _build_gather()(xi32, tpk)
    e2 = e.reshape(_B, _FD)
    xvr = jnp.repeat(xv, _D, axis=1)

    w2q = W2[:_NPAIR, 0] * 0.5
    m26 = (jnp.zeros((_F, _F), jnp.float32)
           .at[_PI, _PJ].set(w2q).at[_PJ, _PI].set(w2q))
    m = jnp.kron(m26, jnp.eye(_D, dtype=jnp.float32))

    return _build_dense()(
        e2, xvr, xv, m, W1,
        b1.reshape(1, _H),
        W2[_NPAIR:, 0].reshape(1, _H),
        b2.reshape(1, 1),
    )
